# baseline (speedup 1.0000x reference)
import jax
import jax.numpy as jnp
from jax import lax
from jax.experimental import pallas as pl
from jax.experimental.pallas import tpu as pltpu

V = 100000
EMB = 200
H = 512
L = 3
B = 64
S = 20
G = 4 * H


def _lstm_body(idx_ref, emb_hbm, fcwt, fcb, wih0, whh0, b0, wih1, whh1, b1,
               wih2, whh2, b2, hn, cn, emb_vmem, xbuf, gbuf, sem):
    NSEM = 8
    CH = (S * B) // NSEM

    def issue(j, _):
        for u in range(NSEM):
            i = u * CH + j
            pltpu.make_async_copy(emb_hbm.at[pl.ds(idx_ref[i], 1)],
                                  emb_vmem.at[pl.ds(i, 1)],
                                  sem.at[u]).start(priority=u % 2)
        return 0

    lax.fori_loop(0, CH, issue, 0)
    for u in range(NSEM):
        pltpu.make_async_copy(emb_hbm.at[pl.ds(0, CH)],
                              emb_vmem.at[pl.ds(u * CH, CH)],
                              sem.at[u]).wait()

    xbuf[:] = jnp.dot(emb_vmem[:], fcwt[:],
                      preferred_element_type=jnp.float32) + fcb[:]
    layers = ((wih0, whh0, b0), (wih1, whh1, b1), (wih2, whh2, b2))
    for l, (wih, whh, bias) in enumerate(layers):
        gbuf[:] = jnp.dot(xbuf[:], wih[:],
                          preferred_element_type=jnp.float32) + bias[:]
        z = jnp.zeros((B, H), jnp.float32)
        h, c = z, z
        for t in range(S):
            g = gbuf[t * B:(t + 1) * B, :] + jnp.dot(
                h.astype(jnp.bfloat16), whh[:],
                preferred_element_type=jnp.float32)
            i = jax.nn.sigmoid(g[:, 0:H])
            f = jax.nn.sigmoid(g[:, H:2 * H])
            gg = jnp.tanh(g[:, 2 * H:3 * H])
            o = jax.nn.sigmoid(g[:, 3 * H:4 * H])
            c = f * c + i * gg
            h = o * jnp.tanh(c)
            if l < L - 1:
                xbuf[t * B:(t + 1) * B, :] = h
        hn[l] = h
        cn[l] = c


def _lstm_call(idx, embedding, fcwt, fcb, layer_args, interpret=False):
    vspec = pl.BlockSpec(memory_space=pltpu.VMEM)
    return pl.pallas_call(
        _lstm_body,
        in_specs=[pl.BlockSpec(memory_space=pltpu.SMEM),
                  pl.BlockSpec(memory_space=pl.ANY)] + [vspec] * 11,
        out_shape=(jax.ShapeDtypeStruct((L, B, H), jnp.float32),
                   jax.ShapeDtypeStruct((L, B, H), jnp.float32)),
        scratch_shapes=[pltpu.VMEM((S * B, EMB), jnp.float32),
                        pltpu.VMEM((S * B, H), jnp.float32),
                        pltpu.VMEM((S * B, G), jnp.float32),
                        pltpu.SemaphoreType.DMA((8,))],
        interpret=interpret,
    )(idx, embedding, fcwt, fcb, *layer_args)


def kernel(x_input, embedding, fc_W, fc_b,
           W_ih_0, W_hh_0, b_ih_0, b_hh_0,
           W_ih_1, W_hh_1, b_ih_1, b_hh_1,
           W_ih_2, W_hh_2, b_ih_2, b_hh_2):
    idx = x_input.T.reshape(-1).astype(jnp.int32)
    fcwt = fc_W.T
    fcb = fc_b.reshape(1, H)
    layer_args = []
    for (Wi, Wh, bi, bh) in ((W_ih_0, W_hh_0, b_ih_0, b_hh_0),
                             (W_ih_1, W_hh_1, b_ih_1, b_hh_1),
                             (W_ih_2, W_hh_2, b_ih_2, b_hh_2)):
        layer_args += [Wi.T, Wh.T.astype(jnp.bfloat16), (bi + bh).reshape(1, G)]
    h_n, c_n = _lstm_call(idx, embedding, fcwt, fcb, layer_args)
    return (h_n, c_n)


# speedup vs baseline: 1.0100x; 1.0100x over previous
import jax
import jax.numpy as jnp
from jax import lax
from jax.experimental import pallas as pl
from jax.experimental.pallas import tpu as pltpu

V = 100000
EMB = 200
H = 512
L = 3
B = 64
S = 20
G = 4 * H


def _lstm_body(idx_ref, emb_hbm, fcwt, fcb, wih0, whh0, b0, wih1, whh1, b1,
               wih2, whh2, b2, hn, cn, emb_vmem, xbuf, gbuf, sem):
    NSEM = 8
    CH = (S * B) // NSEM

    for i in range(S * B):
        pltpu.make_async_copy(emb_hbm.at[pl.ds(idx_ref[i], 1)],
                              emb_vmem.at[pl.ds(i, 1)],
                              sem.at[i // CH]).start()
    for u in range(NSEM):
        pltpu.make_async_copy(emb_hbm.at[pl.ds(0, CH)],
                              emb_vmem.at[pl.ds(u * CH, CH)],
                              sem.at[u]).wait()

    xbuf[:] = jnp.dot(emb_vmem[:], fcwt[:],
                      preferred_element_type=jnp.float32) + fcb[:]
    layers = ((wih0, whh0, b0), (wih1, whh1, b1), (wih2, whh2, b2))
    for l, (wih, whh, bias) in enumerate(layers):
        gbuf[:] = jnp.dot(xbuf[:], wih[:],
                          preferred_element_type=jnp.float32) + bias[:]
        z = jnp.zeros((B, H), jnp.float32)
        h, c = z, z
        for t in range(S):
            g = gbuf[t * B:(t + 1) * B, :] + jnp.dot(
                h.astype(jnp.bfloat16), whh[:],
                preferred_element_type=jnp.float32)
            i = jax.nn.sigmoid(g[:, 0:H])
            f = jax.nn.sigmoid(g[:, H:2 * H])
            gg = jnp.tanh(g[:, 2 * H:3 * H])
            o = jax.nn.sigmoid(g[:, 3 * H:4 * H])
            c = f * c + i * gg
            h = o * jnp.tanh(c)
            if l < L - 1:
                xbuf[t * B:(t + 1) * B, :] = h
        hn[l] = h
        cn[l] = c


def _lstm_call(idx, embedding, fcwt, fcb, layer_args, interpret=False):
    vspec = pl.BlockSpec(memory_space=pltpu.VMEM)
    return pl.pallas_call(
        _lstm_body,
        in_specs=[pl.BlockSpec(memory_space=pltpu.SMEM),
                  pl.BlockSpec(memory_space=pl.ANY)] + [vspec] * 11,
        out_shape=(jax.ShapeDtypeStruct((L, B, H), jnp.float32),
                   jax.ShapeDtypeStruct((L, B, H), jnp.float32)),
        scratch_shapes=[pltpu.VMEM((S * B, EMB), jnp.float32),
                        pltpu.VMEM((S * B, H), jnp.float32),
                        pltpu.VMEM((S * B, G), jnp.float32),
                        pltpu.SemaphoreType.DMA((8,))],
        interpret=interpret,
    )(idx, embedding, fcwt, fcb, *layer_args)


def kernel(x_input, embedding, fc_W, fc_b,
           W_ih_0, W_hh_0, b_ih_0, b_hh_0,
           W_ih_1, W_hh_1, b_ih_1, b_hh_1,
           W_ih_2, W_hh_2, b_ih_2, b_hh_2):
    idx = x_input.T.reshape(-1).astype(jnp.int32)
    fcwt = fc_W.T
    fcb = fc_b.reshape(1, H)
    layer_args = []
    for (Wi, Wh, bi, bh) in ((W_ih_0, W_hh_0, b_ih_0, b_hh_0),
                             (W_ih_1, W_hh_1, b_ih_1, b_hh_1),
                             (W_ih_2, W_hh_2, b_ih_2, b_hh_2)):
        layer_args += [Wi.T, Wh.T.astype(jnp.bfloat16), (bi + bh).reshape(1, G)]
    h_n, c_n = _lstm_call(idx, embedding, fcwt, fcb, layer_args)
    return (h_n, c_n)
